# baseline (speedup 1.0000x reference)
import jax
import jax.numpy as jnp
from jax.experimental import pallas as pl
from jax.experimental.pallas import tpu as pltpu


def _swish(x):
    h = 0.5 * x
    return h + h * jnp.tanh(h)


_TM = 16


def _body(p_ref, s_ref, v_ref, w1_ref, b1_ref, w2_ref, b2_ref, w3_ref,
          wp_ref, bp_ref, o_ref):
    f32 = jnp.float32
    p = p_ref[...]
    y = jnp.dot(p, w1_ref[...], preferred_element_type=f32) + b1_ref[...]
    h = _swish(y).astype(jnp.bfloat16)
    y = jnp.dot(h, w2_ref[...], preferred_element_type=f32) + b2_ref[...]
    h = _swish(y).astype(jnp.bfloat16)
    ones = jnp.ones((h.shape[0], 1), jnp.bfloat16)
    h = jnp.concatenate([h, ones], axis=1)
    w = _swish(jnp.dot(h, w3_ref[...], preferred_element_type=f32))
    v = v_ref[0]
    n = v.shape[0]
    cin = w.shape[-1]
    u = w.reshape(_TM, n, cin) * v[None]
    s = s_ref[...]
    acc = jnp.concatenate(
        [s[t:t + 1, :] @ u[t] for t in range(_TM)], axis=0)
    o_ref[...] = acc @ wp_ref[...] + bp_ref[...]


def kernel(pairs_ab, values, mask, W1, b1, W2, b2, W3, b3, Wp, bp):
    B, M, N, D = pairs_ab.shape
    Cin = values.shape[-1]
    Cout = Wp.shape[-1]
    H = W1.shape[-1]
    BM = B * M

    vals_masked = jnp.where(mask[:, :, None], values, 0.0)
    s = (jnp.linalg.norm(pairs_ab, axis=-1) < 1.0)
    s = s.astype(jnp.float32).reshape(BM, N)
    p_flat = pairs_ab.reshape(BM * N, D).astype(jnp.bfloat16)
    w3_aug = jnp.concatenate([W3, b3.reshape(1, Cin)], axis=0)
    w1_b = W1.astype(jnp.bfloat16)
    w2_b = W2.astype(jnp.bfloat16)
    w3_b = w3_aug.astype(jnp.bfloat16)

    grid = (BM // _TM,)
    out = pl.pallas_call(
        _body,
        grid=grid,
        in_specs=[
            pl.BlockSpec((_TM * N, D), lambda i: (i, 0)),
            pl.BlockSpec((_TM, N), lambda i: (i, 0)),
            pl.BlockSpec((1, N, Cin), lambda i: (i * _TM // M, 0, 0)),
            pl.BlockSpec((D, H), lambda i: (0, 0)),
            pl.BlockSpec((1, H), lambda i: (0, 0)),
            pl.BlockSpec((H, H), lambda i: (0, 0)),
            pl.BlockSpec((1, H), lambda i: (0, 0)),
            pl.BlockSpec((H + 1, Cin), lambda i: (0, 0)),
            pl.BlockSpec((Cin, Cout), lambda i: (0, 0)),
            pl.BlockSpec((1, Cout), lambda i: (0, 0)),
        ],
        out_specs=pl.BlockSpec((_TM, Cout), lambda i: (i, 0)),
        out_shape=jax.ShapeDtypeStruct((BM, Cout), jnp.float32),
        compiler_params=pltpu.CompilerParams(
            dimension_semantics=("arbitrary",),
        ),
    )(p_flat, s, vals_masked, w1_b, b1.reshape(1, H), w2_b, b2.reshape(1, H),
      w3_b, Wp, bp.reshape(1, Cout))

    out = jnp.where(mask.reshape(BM, 1), out, bp[None, :]).reshape(B, M, Cout)
    return (pairs_ab, out, mask)


# speedup vs baseline: 1.3734x; 1.3734x over previous
import jax
import jax.numpy as jnp
from jax.experimental import pallas as pl
from jax.experimental.pallas import tpu as pltpu


def _swish(x):
    h = 0.5 * x
    return h + h * jnp.tanh(h)


_TM = 16


def _body(p_ref, s_ref, v_ref, w1_ref, b1_ref, w2_ref, b2_ref, w3_ref,
          wp_ref, bp_ref, o_ref):
    p = p_ref[...]
    h = _swish(p @ w1_ref[...] + b1_ref[...])
    h = _swish(h @ w2_ref[...] + b2_ref[...])
    ones = jnp.ones((h.shape[0], 1), jnp.float32)
    h = jnp.concatenate([h, ones], axis=1)
    w = _swish(h @ w3_ref[...])
    v = v_ref[0]
    n = v.shape[0]
    cin = w.shape[-1]
    u = w.reshape(_TM, n, cin) * v[None]
    s = s_ref[...]
    acc = jnp.concatenate(
        [s[t:t + 1, :] @ u[t] for t in range(_TM)], axis=0)
    o_ref[...] = acc @ wp_ref[...] + bp_ref[...]


def kernel(pairs_ab, values, mask, W1, b1, W2, b2, W3, b3, Wp, bp):
    B, M, N, D = pairs_ab.shape
    Cin = values.shape[-1]
    Cout = Wp.shape[-1]
    H = W1.shape[-1]
    BM = B * M

    vals_masked = jnp.where(mask[:, :, None], values, 0.0)
    p_flat = pairs_ab.reshape(BM * N, D)
    s = (jnp.linalg.norm(p_flat, axis=-1) < 1.0)
    s = s.astype(jnp.float32).reshape(BM, N)
    w3_aug = jnp.concatenate([W3, b3.reshape(1, Cin)], axis=0)

    grid = (BM // _TM,)
    out = pl.pallas_call(
        _body,
        grid=grid,
        in_specs=[
            pl.BlockSpec((_TM * N, D), lambda i: (i, 0)),
            pl.BlockSpec((_TM, N), lambda i: (i, 0)),
            pl.BlockSpec((1, N, Cin), lambda i: (i * _TM // M, 0, 0)),
            pl.BlockSpec((D, H), lambda i: (0, 0)),
            pl.BlockSpec((1, H), lambda i: (0, 0)),
            pl.BlockSpec((H, H), lambda i: (0, 0)),
            pl.BlockSpec((1, H), lambda i: (0, 0)),
            pl.BlockSpec((H + 1, Cin), lambda i: (0, 0)),
            pl.BlockSpec((Cin, Cout), lambda i: (0, 0)),
            pl.BlockSpec((1, Cout), lambda i: (0, 0)),
        ],
        out_specs=pl.BlockSpec((_TM, Cout), lambda i: (i, 0)),
        out_shape=jax.ShapeDtypeStruct((BM, Cout), jnp.float32),
        compiler_params=pltpu.CompilerParams(
            dimension_semantics=("arbitrary",),
        ),
    )(p_flat, s, vals_masked, W1, b1.reshape(1, H), W2, b2.reshape(1, H),
      w3_aug, Wp, bp.reshape(1, Cout))

    out = jnp.where(mask.reshape(BM, 1), out, bp[None, :]).reshape(B, M, Cout)
    return (pairs_ab, out, mask)
